# baseline (speedup 1.0000x reference)
import functools

import jax
import jax.numpy as jnp
from jax import lax
from jax.experimental import pallas as pl
from jax.experimental.pallas import tpu as pltpu
from jax.experimental.pallas import tpu_sc as plsc

N_FACTORS = 64
BATCH = 16384
_LANES = 16
_CHUNK = 128
_PITCH = 17
_NBUF = 2


def _make_sc_call():
    info = plsc.get_sparse_core_info()
    nc, ns = info.num_cores, info.num_subcores
    nw = nc * ns
    rows = BATCH // nw
    n_chunks = rows // _CHUNK
    groups_per_chunk = _CHUNK // _LANES
    width = 2 * N_FACTORS

    mesh = plsc.VectorSubcoreMesh(core_axis_name="c", subcore_axis_name="s")

    @functools.partial(
        pl.kernel,
        mesh=mesh,
        compiler_params=pltpu.CompilerParams(needs_layout_passes=False,
                                             use_tc_tiling_on_sc=False,
                                             disable_bounds_checks=True,
                                             disable_semaphore_checks=True,
                                             skip_device_barrier=True),
        out_type=jax.ShapeDtypeStruct((BATCH,), jnp.float32),
        scratch_types=[
            pltpu.VMEM((rows,), jnp.int32),
            pltpu.VMEM((rows,), jnp.int32),
            pltpu.VMEM((_NBUF, _CHUNK, width), jnp.bfloat16),
            pltpu.VMEM((_NBUF, _CHUNK, width), jnp.bfloat16),
            pltpu.VMEM((_LANES * _PITCH,), jnp.float32),
            pltpu.VMEM((rows,), jnp.float32),
            pltpu.SemaphoreType.DMA,
        ],
    )
    def sc_call(cats_hbm, c_hbm, out_hbm,
                uidx_v, bidx_v, ubuf_v, bbuf_v, stage_v, out_v, sem):
        wid = lax.axis_index("s") * nc + lax.axis_index("c")
        base = wid * rows

        blk0 = wid * n_chunks
        for j in range(n_chunks):
            pltpu.sync_copy(cats_hbm.at[blk0 + j, 0, :],
                            uidx_v.at[pl.ds(j * _CHUNK, _CHUNK)])
            pltpu.sync_copy(cats_hbm.at[blk0 + j, 1, :],
                            bidx_v.at[pl.ds(j * _CHUNK, _CHUNK)])

        def fire(c, buf):
            sl = pl.ds(c * _CHUNK, _CHUNK)
            return (pltpu.async_copy(c_hbm.at[uidx_v.at[sl]],
                                     ubuf_v.at[buf], sem),
                    pltpu.async_copy(c_hbm.at[bidx_v.at[sl]],
                                     bbuf_v.at[buf], sem))

        iota17 = lax.iota(jnp.int32, _LANES) * _PITCH

        def compute(c, buf):
            def group_body(g, carry):
                for r in range(_LANES):
                    row = g * _LANES + r
                    partial = None
                    for j in range(N_FACTORS // 32):
                        uab = ubuf_v[buf, row, pl.ds(j * 32, 32)]
                        bab = bbuf_v[buf, row,
                                     pl.ds(N_FACTORS + j * 32, 32)]
                        u0, u1 = plsc.unpack(
                            uab, format=plsc.PackFormat.INTERLEAVED)
                        b0, b1 = plsc.unpack(
                            bab, format=plsc.PackFormat.INTERLEAVED)
                        prod = u0 * b0 + u1 * b1
                        partial = prod if partial is None else partial + prod
                    stage_v[pl.ds(r * _PITCH, _LANES)] = partial
                acc = plsc.load_gather(stage_v, [iota17])
                for k in range(1, _LANES):
                    acc = acc + plsc.load_gather(stage_v, [iota17 + k])
                out_v[pl.ds(c * _CHUNK + g * _LANES, _LANES)] = acc
                return carry

            lax.fori_loop(0, groups_per_chunk, group_body, 0)

        pending = [fire(0, 0)]
        for c in range(1, n_chunks):
            pending.append(fire(c, c % _NBUF))
            for cp in pending[c - 1]:
                cp.wait()
            compute(c - 1, (c - 1) % _NBUF)
        for cp in pending[n_chunks - 1]:
            cp.wait()
        compute(n_chunks - 1, (n_chunks - 1) % _NBUF)

        pltpu.sync_copy(out_v, out_hbm.at[pl.ds(base, rows)])

    return sc_call


def kernel(cats, conts, U, B):
    del conts
    cats3 = cats.reshape(BATCH // 128, 128, 2).transpose(0, 2, 1)
    c16 = jnp.concatenate([U, B], axis=1).astype(jnp.bfloat16)
    return _make_sc_call()(cats3, c16)


# speedup vs baseline: 1.5954x; 1.5954x over previous
import functools

import jax
import jax.numpy as jnp
from jax import lax
from jax.experimental import pallas as pl
from jax.experimental.pallas import tpu as pltpu
from jax.experimental.pallas import tpu_sc as plsc

N_FACTORS = 64
BATCH = 16384
_LANES = 16
_CHUNK = 128
_PITCH = 17
_NBUF = 2


def _make_sc_call():
    info = plsc.get_sparse_core_info()
    nc, ns = info.num_cores, info.num_subcores
    nw = nc * ns
    rows = BATCH // nw
    n_chunks = rows // _CHUNK
    groups_per_chunk = _CHUNK // _LANES
    width = 2 * N_FACTORS

    mesh = plsc.VectorSubcoreMesh(core_axis_name="c", subcore_axis_name="s")

    @functools.partial(
        pl.kernel,
        mesh=mesh,
        compiler_params=pltpu.CompilerParams(needs_layout_passes=False,
                                             use_tc_tiling_on_sc=False,
                                             disable_bounds_checks=True,
                                             disable_semaphore_checks=True,
                                             skip_device_barrier=True),
        out_type=jax.ShapeDtypeStruct((BATCH,), jnp.float32),
        scratch_types=[
            pltpu.VMEM((rows,), jnp.int32),
            pltpu.VMEM((rows,), jnp.int32),
            pltpu.VMEM((_NBUF, _CHUNK, width), jnp.float32),
            pltpu.VMEM((_NBUF, _CHUNK, width), jnp.float32),
            pltpu.VMEM((_LANES * _PITCH,), jnp.float32),
            pltpu.VMEM((rows,), jnp.float32),
            pltpu.SemaphoreType.DMA,
        ],
    )
    def sc_call(cats_hbm, c_hbm, out_hbm,
                uidx_v, bidx_v, ubuf_v, bbuf_v, stage_v, out_v, sem):
        wid = lax.axis_index("s") * nc + lax.axis_index("c")
        base = wid * rows

        blk0 = wid * n_chunks
        for j in range(n_chunks):
            pltpu.sync_copy(cats_hbm.at[blk0 + j, 0, :],
                            uidx_v.at[pl.ds(j * _CHUNK, _CHUNK)])
            pltpu.sync_copy(cats_hbm.at[blk0 + j, 1, :],
                            bidx_v.at[pl.ds(j * _CHUNK, _CHUNK)])

        def fire(c, buf):
            sl = pl.ds(c * _CHUNK, _CHUNK)
            return (pltpu.async_copy(c_hbm.at[uidx_v.at[sl]],
                                     ubuf_v.at[buf], sem),
                    pltpu.async_copy(c_hbm.at[bidx_v.at[sl]],
                                     bbuf_v.at[buf], sem))

        iota17 = lax.iota(jnp.int32, _LANES) * _PITCH

        def compute(c, buf):
            def group_body(g, carry):
                for r in range(_LANES):
                    row = g * _LANES + r
                    partial = (ubuf_v[buf, row, pl.ds(0, _LANES)]
                               * bbuf_v[buf, row, pl.ds(N_FACTORS, _LANES)])
                    for k in range(1, N_FACTORS // _LANES):
                        partial = partial + (
                            ubuf_v[buf, row, pl.ds(k * _LANES, _LANES)]
                            * bbuf_v[buf, row,
                                     pl.ds(N_FACTORS + k * _LANES, _LANES)])
                    stage_v[pl.ds(r * _PITCH, _LANES)] = partial
                acc = plsc.load_gather(stage_v, [iota17])
                for k in range(1, _LANES):
                    acc = acc + plsc.load_gather(stage_v, [iota17 + k])
                out_v[pl.ds(c * _CHUNK + g * _LANES, _LANES)] = acc
                return carry

            lax.fori_loop(0, groups_per_chunk, group_body, 0)

        pending = [fire(0, 0)]
        for c in range(1, n_chunks):
            pending.append(fire(c, c % _NBUF))
            for cp in pending[c - 1]:
                cp.wait()
            compute(c - 1, (c - 1) % _NBUF)
        for cp in pending[n_chunks - 1]:
            cp.wait()
        compute(n_chunks - 1, (n_chunks - 1) % _NBUF)

        pltpu.sync_copy(out_v, out_hbm.at[pl.ds(base, rows)])

    return sc_call


def kernel(cats, conts, U, B):
    del conts
    cats3 = cats.reshape(BATCH // 128, 128, 2).transpose(0, 2, 1)
    c_tab = jnp.concatenate([U, B], axis=1)
    return _make_sc_call()(cats3, c_tab)
